# Initial kernel scaffold; baseline (speedup 1.0000x reference)
import jax
import jax.numpy as jnp
from jax.experimental import pallas as pl


def kernel(user_inds, item_inds, user_ne_items, user_ne_users, item_ne_users, item_ne_items, user_text_feats, item_text_feats, user_emb, item_emb, node_W1, node_b1, node_W2, node_b2, att_W1, att_b1, att_W2, att_b2, g_W1, g_b1, g_W2, g_b2):
    raise NotImplementedError("write your pallas kernel here")


# speedup vs baseline: 1.8491x; 1.8491x over previous
import functools

import jax
import jax.numpy as jnp
from jax import lax
from jax.experimental import pallas as pl
from jax.experimental.pallas import tpu as pltpu
from jax.experimental.pallas import tpu_sc as plsc

B = 1024
K = 20
H = 64
NB = 8
BB = B // NB
R_DIFF = BB * K
R_SAME = BB * (K + 1)
R_BLK = R_DIFF + R_SAME
RT = NB * R_BLK

NW = 32
RPT = RT // NW
_CHUNKS = [(i * 128, 128) for i in range(RPT // 128)]
if RPT % 128:
    _CHUNKS.append((RPT - RPT % 128, RPT % 128))


def _sc_gather_body(idx_a, idx_b, tab_a, tab_b, out_a, out_b, idx_v, rows_v, sem):
    wid = lax.axis_index("s") * 2 + lax.axis_index("c")
    base = wid * RPT
    for idx_hbm, tab, out in ((idx_a, tab_a, out_a), (idx_b, tab_b, out_b)):
        pltpu.sync_copy(idx_hbm.at[pl.ds(base, RPT)], idx_v)
        handles = []
        for off, sz in _CHUNKS:
            handles.append(
                pltpu.async_copy(
                    tab.at[idx_v.at[pl.ds(off, sz)]],
                    rows_v.at[pl.ds(off, sz)],
                    sem,
                ))
        for h in handles:
            h.wait()
        pltpu.sync_copy(rows_v, out.at[pl.ds(base, RPT)])


_sc_gather = pl.kernel(
    _sc_gather_body,
    mesh=plsc.VectorSubcoreMesh(core_axis_name="c", subcore_axis_name="s"),
    compiler_params=pltpu.CompilerParams(use_tc_tiling_on_sc=False),
    out_type=[
        jax.ShapeDtypeStruct((RT, H), jnp.float32),
        jax.ShapeDtypeStruct((RT, H), jnp.float32),
    ],
    scratch_types=[
        pltpu.VMEM((RPT,), jnp.int32),
        pltpu.VMEM((RPT, H), jnp.float32),
        pltpu.SemaphoreType.DMA,
    ],
)


def _tc_body(tu, ti, ga, gb, nW1, nW2, gW1, gW2, nb1, nb2, gb1, gb2, out):
    w1t = nW1[:H, :]
    w1e = nW1[H:, :]
    w2 = nW2[...]
    b1 = nb1[0, :]
    b2 = nb2[0, :]

    def node_side(t, e_diff, e_same):
        xd = t[:, :K, :].reshape(R_DIFF, H)
        xs = t[:, K:2 * K + 1, :].reshape(R_SAME, H)
        hd = jnp.maximum(
            jnp.dot(xd, w1t, preferred_element_type=jnp.float32)
            + jnp.dot(e_diff, w1e, preferred_element_type=jnp.float32) + b1, 0.0)
        hd = jnp.maximum(
            jnp.dot(hd, w2, preferred_element_type=jnp.float32) + b2, 0.0)
        hs = jnp.maximum(
            jnp.dot(xs, w1t, preferred_element_type=jnp.float32)
            + jnp.dot(e_same, w1e, preferred_element_type=jnp.float32) + b1, 0.0)
        hs = jnp.maximum(
            jnp.dot(hs, w2, preferred_element_type=jnp.float32) + b2, 0.0)
        hd3 = hd.reshape(BB, K, H)
        hs3 = hs.reshape(BB, K + 1, H)
        this = hs3[:, K, :]
        pref = hd3.sum(axis=1) + hs3.sum(axis=1) - this
        return this, pref

    tu_ = tu[...]
    ti_ = ti[...]
    ga_ = ga[...]
    gb_ = gb[...]
    this_u, pref_u = node_side(tu_, ga_[:R_DIFF, :], gb_[R_DIFF:, :])
    this_i, pref_i = node_side(ti_, gb_[:R_DIFF, :], ga_[R_DIFF:, :])

    g1t = gW1[:H, :]
    g1p = gW1[H:, :]
    g2 = gW2[...]
    c1 = gb1[0, :]
    c2 = gb2[0, :]

    def transform(this, pref):
        h = jnp.maximum(
            jnp.dot(this, g1t, preferred_element_type=jnp.float32)
            + jnp.dot(pref, g1p, preferred_element_type=jnp.float32) + c1, 0.0)
        return jnp.maximum(
            jnp.dot(h, g2, preferred_element_type=jnp.float32) + c2, 0.0)

    u = transform(this_u, pref_u)
    v = transform(this_i, pref_i)
    out[...] = jnp.sum(u * v, axis=1)


def kernel(user_inds, item_inds, user_ne_items, user_ne_users, item_ne_users,
           item_ne_items, user_text_feats, item_text_feats, user_emb, item_emb,
           node_W1, node_b1, node_W2, node_b2, att_W1, att_b1, att_W2, att_b2,
           g_W1, g_b1, g_W2, g_b2):
    ism = jnp.concatenate([item_ne_items, item_inds[:, None]], axis=1)
    usm = jnp.concatenate([user_ne_users, user_inds[:, None]], axis=1)
    idx_a = jnp.concatenate(
        [user_ne_items.reshape(NB, R_DIFF), ism.reshape(NB, R_SAME)],
        axis=1).reshape(RT)
    idx_b = jnp.concatenate(
        [item_ne_users.reshape(NB, R_DIFF), usm.reshape(NB, R_SAME)],
        axis=1).reshape(RT)

    ga, gb = _sc_gather(idx_a, idx_b, item_emb, user_emb)

    wspec = pl.BlockSpec((2 * H, H), lambda i: (0, 0))
    w2spec = pl.BlockSpec((H, H), lambda i: (0, 0))
    bspec = pl.BlockSpec((1, H), lambda i: (0, 0))
    out = pl.pallas_call(
        _tc_body,
        grid=(NB,),
        in_specs=[
            pl.BlockSpec((BB, 3 * K + 1, H), lambda i: (i, 0, 0)),
            pl.BlockSpec((BB, 3 * K + 1, H), lambda i: (i, 0, 0)),
            pl.BlockSpec((R_BLK, H), lambda i: (i, 0)),
            pl.BlockSpec((R_BLK, H), lambda i: (i, 0)),
            wspec, w2spec, wspec, w2spec, bspec, bspec, bspec, bspec,
        ],
        out_specs=pl.BlockSpec((BB,), lambda i: (i,)),
        out_shape=jax.ShapeDtypeStruct((B,), jnp.float32),
        compiler_params=pltpu.CompilerParams(
            dimension_semantics=("arbitrary",)),
    )(user_text_feats, item_text_feats, ga, gb,
      node_W1, node_W2, g_W1, g_W2,
      node_b1.reshape(1, H), node_b2.reshape(1, H),
      g_b1.reshape(1, H), g_b2.reshape(1, H))
    return out
